# Initial kernel scaffold; baseline (speedup 1.0000x reference)
import jax
import jax.numpy as jnp
from jax.experimental import pallas as pl


def kernel(emb_user, emb_item, relation_embedding, edge_index, edge_neg_head, edge_neg_tail):
    raise NotImplementedError("write your pallas kernel here")


# speedup vs baseline: 1.2809x; 1.2809x over previous
import functools

import jax
import jax.numpy as jnp
from jax import lax
from jax.experimental import pallas as pl
from jax.experimental.pallas import tpu as pltpu
from jax.experimental.pallas import tpu_sc as plsc

N = 100000
D = 64
E = 1024
NEG = 4

B = E + E * NEG + E
NUM_CORES = 2
NUM_SUBCORES = 16
NW = NUM_CORES * NUM_SUBCORES
B_PER_W = B // NW
CHUNK = 96
N_CHUNKS = B_PER_W // CHUNK


def _sc_gather_body(user_hbm, item_hbm, uidx_hbm, iidx_hbm,
                    out_u, out_i, idx_v, rows_v, sem):
    wid = lax.axis_index("s") * NUM_CORES + lax.axis_index("c")
    base = wid * B_PER_W
    for tbl, idxs, out in ((user_hbm, uidx_hbm, out_u),
                           (item_hbm, iidx_hbm, out_i)):
        for c in range(N_CHUNKS):
            off = base + c * CHUNK
            pltpu.sync_copy(idxs.at[pl.ds(off, CHUNK)], idx_v)
            pltpu.async_copy(tbl.at[idx_v], rows_v, sem).wait()
            pltpu.sync_copy(rows_v, out.at[pl.ds(off, CHUNK)])


def _half(g2, par_ref):
    mask = par_ref[...] == 0
    return jnp.where(mask, g2[:, :D], g2[:, D:])


def _tc_scores_body(gu_ref, gi_ref, k_ref, paru_ref, pari_ref,
                    pos_ref, head_ref, tail_ref):
    gu = _half(gu_ref[...], paru_ref)
    gi = _half(gi_ref[...], pari_ref)
    rel = k_ref[...]
    g_pos = gu[:E]
    g_head = gu[E:E + E * NEG]
    g_head0 = gu[E + E * NEG:]
    g_item = gi[:E]
    g_tail = gi[E:E + E * NEG]
    g_tail0 = gi[E + E * NEG:]

    s_b = jnp.sum(g_item, axis=0, keepdims=True)
    s_h = jnp.sum(g_head0, axis=0, keepdims=True)
    s_t = jnp.sum(g_tail, axis=0, keepdims=True)

    ct = (((1,), (1,)), ((), ()))
    u_pos = lax.dot_general(s_b, rel, ct, preferred_element_type=jnp.float32)
    u_head = float(NEG) * lax.dot_general(s_h, rel, ct,
                                          preferred_element_type=jnp.float32)
    u_tail = lax.dot_general(s_t, rel, ct, preferred_element_type=jnp.float32)

    pos_ref[...] = lax.dot_general(g_pos, u_pos, ct,
                                   preferred_element_type=jnp.float32)
    head_ref[...] = lax.dot_general(g_head, u_head, ct,
                                    preferred_element_type=jnp.float32)
    tail_ref[...] = lax.dot_general(g_tail0, u_tail, ct,
                                    preferred_element_type=jnp.float32)


def _run_sc_gather(user2, item2, uidx2, iidx2):
    mesh = plsc.VectorSubcoreMesh(core_axis_name="c", subcore_axis_name="s")
    gather = functools.partial(
        pl.kernel,
        mesh=mesh,
        out_type=[jax.ShapeDtypeStruct((B, 2 * D), jnp.float32),
                  jax.ShapeDtypeStruct((B, 2 * D), jnp.float32)],
        scratch_types=[
            pltpu.VMEM((CHUNK,), jnp.int32),
            pltpu.VMEM((CHUNK, 2 * D), jnp.float32),
            pltpu.SemaphoreType.DMA,
        ],
    )(_sc_gather_body)
    return gather(user2, item2, uidx2, iidx2)


def kernel(emb_user, emb_item, relation_embedding, edge_index,
           edge_neg_head, edge_neg_tail):
    uidx = jnp.concatenate([edge_index[0],
                            edge_neg_head.reshape(-1),
                            edge_neg_head[:, 0]]).astype(jnp.int32)
    iidx = jnp.concatenate([edge_index[1],
                            edge_neg_tail.reshape(-1),
                            edge_neg_tail[:, 0]]).astype(jnp.int32)

    user2 = emb_user.reshape(N // 2, 2 * D)
    item2 = emb_item.reshape(N // 2, 2 * D)
    gu2, gi2 = _run_sc_gather(user2, item2, uidx >> 1, iidx >> 1)

    par_u = (uidx & 1).reshape(B, 1)
    par_i = (iidx & 1).reshape(B, 1)
    pos, head, tail0 = pl.pallas_call(
        _tc_scores_body,
        out_shape=[jax.ShapeDtypeStruct((E, 1), jnp.float32),
                   jax.ShapeDtypeStruct((E * NEG, 1), jnp.float32),
                   jax.ShapeDtypeStruct((E, 1), jnp.float32)],
    )(gu2, gi2, relation_embedding[0], par_u, par_i)

    score_pos = pos.reshape(E)
    score_head = head.reshape(E * NEG)
    score_tail = jnp.repeat(tail0.reshape(E), NEG)
    return (score_pos, score_head, score_tail)


# speedup vs baseline: 1.2959x; 1.0117x over previous
import functools

import jax
import jax.numpy as jnp
from jax import lax
from jax.experimental import pallas as pl
from jax.experimental.pallas import tpu as pltpu
from jax.experimental.pallas import tpu_sc as plsc

N = 100000
D = 64
E = 1024
NEG = 4

B = E + E * NEG + E
NUM_CORES = 2
NUM_SUBCORES = 16
NW = NUM_CORES * NUM_SUBCORES
B_PER_W = B // NW
CHUNK = 96
N_CHUNKS = B_PER_W // CHUNK


def _sc_gather_body(user_hbm, item_hbm, uidx_hbm, iidx_hbm,
                    out_u, out_i, idx_v, rows_v, sem):
    wid = lax.axis_index("s") * NUM_CORES + lax.axis_index("c")
    base = wid * B_PER_W
    for tbl, idxs, out in ((user_hbm, uidx_hbm, out_u),
                           (item_hbm, iidx_hbm, out_i)):
        for c in range(N_CHUNKS):
            off = base + c * CHUNK
            pltpu.sync_copy(idxs.at[pl.ds(off, CHUNK)], idx_v)
            pltpu.async_copy(tbl.at[idx_v], rows_v, sem).wait()
            pltpu.sync_copy(rows_v, out.at[pl.ds(off, CHUNK)])


def _tc_scores_body(gu_ref, gi_ref, k_ref, pos_ref, head_ref, tail_ref):
    gu = gu_ref[...]
    gi = gi_ref[...]
    rel = k_ref[...]
    g_pos = gu[:E]
    g_head = gu[E:E + E * NEG]
    g_head0 = gu[E + E * NEG:]
    g_item = gi[:E]
    g_tail = gi[E:E + E * NEG]
    g_tail0 = gi[E + E * NEG:]

    s_b = jnp.sum(g_item, axis=0, keepdims=True)
    s_h = jnp.sum(g_head0, axis=0, keepdims=True)
    s_t = jnp.sum(g_tail, axis=0, keepdims=True)

    ct = (((1,), (1,)), ((), ()))
    u_pos = lax.dot_general(s_b, rel, ct, preferred_element_type=jnp.float32)
    u_head = float(NEG) * lax.dot_general(s_h, rel, ct,
                                          preferred_element_type=jnp.float32)
    u_tail = lax.dot_general(s_t, rel, ct, preferred_element_type=jnp.float32)

    pos_ref[...] = lax.dot_general(g_pos, u_pos, ct,
                                   preferred_element_type=jnp.float32)
    head_ref[...] = lax.dot_general(g_head, u_head, ct,
                                    preferred_element_type=jnp.float32)
    tail_ref[...] = lax.dot_general(g_tail0, u_tail, ct,
                                    preferred_element_type=jnp.float32)


def _run_sc_gather(emb_user, emb_item, uidx, iidx):
    mesh = plsc.VectorSubcoreMesh(core_axis_name="c", subcore_axis_name="s")
    gather = functools.partial(
        pl.kernel,
        mesh=mesh,
        out_type=[jax.ShapeDtypeStruct((B, D), jnp.float32),
                  jax.ShapeDtypeStruct((B, D), jnp.float32)],
        scratch_types=[
            pltpu.VMEM((CHUNK,), jnp.int32),
            pltpu.VMEM((CHUNK, D), jnp.float32),
            pltpu.SemaphoreType.DMA,
        ],
        compiler_params=pltpu.CompilerParams(use_tc_tiling_on_sc=False),
    )(_sc_gather_body)
    return gather(emb_user, emb_item, uidx, iidx)


def kernel(emb_user, emb_item, relation_embedding, edge_index,
           edge_neg_head, edge_neg_tail):
    uidx = jnp.concatenate([edge_index[0],
                            edge_neg_head.reshape(-1),
                            edge_neg_head[:, 0]]).astype(jnp.int32)
    iidx = jnp.concatenate([edge_index[1],
                            edge_neg_tail.reshape(-1),
                            edge_neg_tail[:, 0]]).astype(jnp.int32)

    gu, gi = _run_sc_gather(emb_user, emb_item, uidx, iidx)

    pos, head, tail0 = pl.pallas_call(
        _tc_scores_body,
        out_shape=[jax.ShapeDtypeStruct((E, 1), jnp.float32),
                   jax.ShapeDtypeStruct((E * NEG, 1), jnp.float32),
                   jax.ShapeDtypeStruct((E, 1), jnp.float32)],
    )(gu, gi, relation_embedding[0])

    score_pos = pos.reshape(E)
    score_head = head.reshape(E * NEG)
    score_tail = jnp.repeat(tail0.reshape(E), NEG)
    return (score_pos, score_head, score_tail)


# speedup vs baseline: 1.6333x; 1.2604x over previous
import functools

import jax
import jax.numpy as jnp
from jax import lax
from jax.experimental import pallas as pl
from jax.experimental.pallas import tpu as pltpu
from jax.experimental.pallas import tpu_sc as plsc

N = 100000
D = 64
E = 1024
NEG = 4

B = E + E * NEG + E
NUM_CORES = 2
NUM_SUBCORES = 16
NW = NUM_CORES * NUM_SUBCORES
B_PER_W = B // NW
KFIRE = 16


def _sc_gather_body(user_hbm, item_hbm, uidx_hbm, iidx_hbm,
                    out_u, out_i, idx_v, rows_v, sem):
    wid = lax.axis_index("s") * NUM_CORES + lax.axis_index("c")
    base = wid * B_PER_W
    for tbl, idxs, out in ((user_hbm, uidx_hbm, out_u),
                           (item_hbm, iidx_hbm, out_i)):
        pltpu.sync_copy(idxs.at[pl.ds(base, B_PER_W)], idx_v)

        def chunk(c0, tbl=tbl):
            vec = idx_v[pl.ds(c0 * KFIRE, KFIRE)]
            descs = []
            for i in range(KFIRE):
                descs.append(pltpu.async_copy(
                    tbl.at[pl.ds(vec[i], 1)],
                    rows_v.at[pl.ds(c0 * KFIRE + i, 1)],
                    sem))
            for d in descs:
                d.wait()

        pl.loop(0, B_PER_W // KFIRE)(chunk)
        pltpu.sync_copy(rows_v, out.at[pl.ds(base, B_PER_W)])


def _tc_scores_body(gu_ref, gi_ref, k_ref, pos_ref, head_ref, tail_ref):
    gu = gu_ref[...]
    gi = gi_ref[...]
    rel = k_ref[...]
    g_pos = gu[:E]
    g_head = gu[E:E + E * NEG]
    g_head0 = gu[E + E * NEG:]
    g_item = gi[:E]
    g_tail = gi[E:E + E * NEG]
    g_tail0 = gi[E + E * NEG:]

    s_b = jnp.sum(g_item, axis=0, keepdims=True)
    s_h = jnp.sum(g_head0, axis=0, keepdims=True)
    s_t = jnp.sum(g_tail, axis=0, keepdims=True)

    ct = (((1,), (1,)), ((), ()))
    u_pos = lax.dot_general(s_b, rel, ct, preferred_element_type=jnp.float32)
    u_head = float(NEG) * lax.dot_general(s_h, rel, ct,
                                          preferred_element_type=jnp.float32)
    u_tail = lax.dot_general(s_t, rel, ct, preferred_element_type=jnp.float32)

    pos_ref[...] = lax.dot_general(g_pos, u_pos, ct,
                                   preferred_element_type=jnp.float32)
    head_ref[...] = lax.dot_general(g_head, u_head, ct,
                                    preferred_element_type=jnp.float32)
    tail_ref[...] = lax.dot_general(g_tail0, u_tail, ct,
                                    preferred_element_type=jnp.float32)


def _run_sc_gather(emb_user, emb_item, uidx, iidx):
    mesh = plsc.VectorSubcoreMesh(core_axis_name="c", subcore_axis_name="s")
    gather = functools.partial(
        pl.kernel,
        mesh=mesh,
        out_type=[jax.ShapeDtypeStruct((B, D), jnp.float32),
                  jax.ShapeDtypeStruct((B, D), jnp.float32)],
        scratch_types=[
            pltpu.VMEM((B_PER_W,), jnp.int32),
            pltpu.VMEM((B_PER_W, D), jnp.float32),
            pltpu.SemaphoreType.DMA,
        ],
    )(_sc_gather_body)
    return gather(emb_user, emb_item, uidx, iidx)


def kernel(emb_user, emb_item, relation_embedding, edge_index,
           edge_neg_head, edge_neg_tail):
    uidx = jnp.concatenate([edge_index[0],
                            edge_neg_head.reshape(-1),
                            edge_neg_head[:, 0]]).astype(jnp.int32)
    iidx = jnp.concatenate([edge_index[1],
                            edge_neg_tail.reshape(-1),
                            edge_neg_tail[:, 0]]).astype(jnp.int32)

    gu, gi = _run_sc_gather(emb_user, emb_item, uidx, iidx)

    pos, head, tail0 = pl.pallas_call(
        _tc_scores_body,
        out_shape=[jax.ShapeDtypeStruct((E, 1), jnp.float32),
                   jax.ShapeDtypeStruct((E * NEG, 1), jnp.float32),
                   jax.ShapeDtypeStruct((E, 1), jnp.float32)],
    )(gu, gi, relation_embedding[0])

    score_pos = pos.reshape(E)
    score_head = head.reshape(E * NEG)
    score_tail = jnp.repeat(tail0.reshape(E), NEG)
    return (score_pos, score_head, score_tail)
